# baseline (speedup 1.0000x reference)
import functools

import jax
import jax.numpy as jnp
from jax import lax
from jax.experimental import pallas as pl
from jax.experimental.pallas import tpu as pltpu
from jax.experimental.pallas import tpu_sc as plsc

_N = 10000
_K = 16
_D = 128
_DP = _D // 2
_H = 256
_E = _N * _K


_P = 5
_EP = _E // _P
_CH = 128
_NCHUNK = _EP // _CH
_NC = 2
_NS = 16
_NW = _NC * _NS


def _pack_pairs(rows, packed, t):
    for c in range(_D // 16):
        a = rows[2 * t, pl.ds(16 * c, 16)]
        b = rows[2 * t + 1, pl.ds(16 * c, 16)]
        v = plsc.pack(a, b, format=plsc.PackFormat.INTERLEAVED)
        packed[t, pl.ds(16 * c, 16)] = plsc.bitcast(v, jnp.int32)


def _sc_gather_kernel(x_hbm, recv_hbm, send_hbm, rec_out, snd_out,
                      ridx, rrows, rpack, sidx, srows, spack, rsem, ssem):
    wid = lax.axis_index("s") * _NC + lax.axis_index("c")
    nt = (_NCHUNK - wid + _NW - 1) // _NW

    def body(t, carry):
        base = (wid + t * _NW) * _CH
        base2 = (wid + t * _NW) * (_CH // 2)
        pltpu.sync_copy(recv_hbm.at[pl.ds(base, _CH)], ridx)
        pltpu.sync_copy(send_hbm.at[pl.ds(base, _CH)], sidx)
        r1 = pltpu.async_copy(x_hbm.at[ridx], rrows, rsem)
        r2 = pltpu.async_copy(x_hbm.at[sidx], srows, ssem)
        r1.wait()
        r2.wait()

        def pack_body(u, c2):
            _pack_pairs(rrows, rpack, u)
            _pack_pairs(srows, spack, u)
            return c2

        lax.fori_loop(0, _CH // 2, pack_body, 0)
        pltpu.sync_copy(rpack, rec_out.at[pl.ds(base2, _CH // 2)])
        pltpu.sync_copy(spack, snd_out.at[pl.ds(base2, _CH // 2)])
        return carry

    lax.fori_loop(0, nt, body, 0)


@functools.cache
def _sc_gather():
    return pl.kernel(
        _sc_gather_kernel,
        mesh=plsc.VectorSubcoreMesh(core_axis_name="c", subcore_axis_name="s"),
        compiler_params=pltpu.CompilerParams(needs_layout_passes=False),
        out_type=(
            jax.ShapeDtypeStruct((_EP // 2, _D), jnp.int32),
            jax.ShapeDtypeStruct((_EP // 2, _D), jnp.int32),
        ),
        scratch_types=[
            pltpu.VMEM((_CH,), jnp.int32),
            pltpu.VMEM((_CH, _D), jnp.float32),
            pltpu.VMEM((_CH // 2, _D), jnp.int32),
            pltpu.VMEM((_CH,), jnp.int32),
            pltpu.VMEM((_CH, _D), jnp.float32),
            pltpu.VMEM((_CH // 2, _D), jnp.int32),
            pltpu.SemaphoreType.DMA,
            pltpu.SemaphoreType.DMA,
        ],
    )


_NB = 400
_BE = _NB * _K
_GRID = _N // _NB
_SPP = _GRID // _P
_NNB = 1000


def _ln(h, g, bt):
    mu = jnp.mean(h, axis=-1, keepdims=True)
    var = jnp.mean((h - mu) * (h - mu), axis=-1, keepdims=True)
    return g * ((h - mu) * lax.rsqrt(var + 1e-5)) + bt


def _unpack_rows(p):
    bf = jnp.bfloat16
    lo = lax.bitcast_convert_type(p << 16, jnp.float32).astype(bf)
    hi = lax.bitcast_convert_type(p & jnp.int32(-65536), jnp.float32).astype(bf)
    return lo, hi


def _tc_kernel(rec, snd, ea,
               w1r, w1s, w1e, b1, w2, b2, w3, b3, g, bt,
               m_out, s_out):
    f32 = jnp.float32
    bf = jnp.bfloat16
    rlo, rhi = _unpack_rows(rec[...])
    slo, shi = _unpack_rows(snd[...])
    h_e = (jnp.dot(rlo, w1r[...], preferred_element_type=f32)
           + jnp.dot(slo, w1s[...], preferred_element_type=f32))
    h_o = (jnp.dot(rhi, w1r[...], preferred_element_type=f32)
           + jnp.dot(shi, w1s[...], preferred_element_type=f32))
    h = jnp.concatenate([h_e[:, None, :], h_o[:, None, :]],
                        axis=1).reshape(_BE, _H)
    h = h + jnp.dot(ea[...].astype(bf), w1e[...], preferred_element_type=f32)
    h = jax.nn.relu(h + b1[...])
    h = jax.nn.relu(jnp.dot(h.astype(bf), w2[...], preferred_element_type=f32)
                    + b2[...])
    m = jnp.dot(h.astype(bf), w3[...], preferred_element_type=f32) + b3[...]
    mln = _ln(m, g[...], bt[...])
    m_out[...] = mln
    grp = (lax.broadcasted_iota(jnp.int32, (_D, 8), 0) // 16
           == lax.broadcasted_iota(jnp.int32, (_D, 8), 1))
    s_out[...] = jnp.dot(mln, grp.astype(f32), preferred_element_type=f32)


def _node_kernel(xin, nw1, nb1, nw2, nb2, nw3, nb3, ng, nbt, x_out):
    f32 = jnp.float32
    bf = jnp.bfloat16
    h = jax.nn.relu(
        jnp.dot(xin[...].astype(bf), nw1[...], preferred_element_type=f32)
        + nb1[...])
    h = jax.nn.relu(jnp.dot(h.astype(bf), nw2[...], preferred_element_type=f32)
                    + nb2[...])
    y = jnp.dot(h.astype(bf), nw3[...], preferred_element_type=f32) + nb3[...]
    x_out[...] = _ln(y, ng[...], nbt[...])


def _const(shape):
    return pl.BlockSpec(shape, lambda i: tuple(0 for _ in shape))


def _tc_edge_part(part, rec_p, snd_p, ea, ws, m_prev, s_prev,
                  interpret=False):
    (w1r, w1s, w1e, b1, w2, b2, w3, b3, g, bt,
     nw1, nb1, nw2, nb2, nw3, nb3, ng, nbt) = ws
    off = part * _SPP
    part_spec = pl.BlockSpec((_BE // 2, _D), lambda i: (i, 0))
    full_spec = pl.BlockSpec((_BE, _D), lambda i: (i + off, 0))
    in_specs = [
        part_spec, part_spec, full_spec,
        _const((_D, _H)), _const((_D, _H)), _const((_D, _H)),
        _const((1, _H)),
        _const((_H, _H)), _const((1, _H)),
        _const((_H, _D)), _const((1, _D)),
        _const((1, _D)), _const((1, _D)),
    ]
    args = [rec_p, snd_p, ea, w1r, w1s, w1e, b1, w2, b2, w3, b3, g, bt]
    aliases = {}
    if m_prev is not None:
        in_specs += [pl.BlockSpec(memory_space=pl.ANY),
                     pl.BlockSpec(memory_space=pl.ANY)]
        args += [m_prev, s_prev]
        aliases = {13: 0, 14: 1}

    def body(*refs):
        _tc_kernel(*refs[:13], refs[-2], refs[-1])

    m, s = pl.pallas_call(
        body,
        grid=(_SPP,),
        in_specs=in_specs,
        out_specs=[
            pl.BlockSpec((_BE, _D), lambda i: (i + off, 0)),
            pl.BlockSpec((_BE, 8), lambda i: (i + off, 0)),
        ],
        out_shape=[
            jax.ShapeDtypeStruct((_E, _D), jnp.float32),
            jax.ShapeDtypeStruct((_E, 8), jnp.float32),
        ],
        input_output_aliases=aliases,
        interpret=interpret,
    )(*args)
    return m, s


def _node_call(s, ws, interpret=False):
    (w1r, w1s, w1e, b1, w2, b2, w3, b3, g, bt,
     nw1, nb1, nw2, nb2, nw3, nb3, ng, nbt) = ws
    xin = s.reshape(_N, _D)
    x_out = pl.pallas_call(
        _node_kernel,
        grid=(_N // _NNB,),
        in_specs=[
            pl.BlockSpec((_NNB, _D), lambda i: (i, 0)),
            _const((_D, _H)), _const((1, _H)),
            _const((_H, _H)), _const((1, _H)),
            _const((_H, _D)), _const((1, _D)),
            _const((1, _D)), _const((1, _D)),
        ],
        out_specs=pl.BlockSpec((_NNB, _D), lambda i: (i, 0)),
        out_shape=jax.ShapeDtypeStruct((_N, _D), jnp.float32),
        interpret=interpret,
    )(xin, nw1, nb1, nw2, nb2, nw3, nb3, ng, nbt)
    return x_out


def _prep_weights(eW1, eb1, eW2, eb2, eW3, eb3, eg, ebt,
                  nW1, nb1, nW2, nb2, nW3, nb3, ng, nbt):
    bf = jnp.bfloat16
    return (eW1[:_D].astype(bf), eW1[_D:2 * _D].astype(bf),
            eW1[2 * _D:].astype(bf),
            eb1.reshape(1, _H), eW2.astype(bf), eb2.reshape(1, _H),
            eW3.astype(bf), eb3.reshape(1, _D),
            eg.reshape(1, _D), ebt.reshape(1, _D),
            nW1.astype(bf), nb1.reshape(1, _H),
            nW2.astype(bf), nb2.reshape(1, _H),
            nW3.astype(bf), nb3.reshape(1, _D),
            ng.reshape(1, _D), nbt.reshape(1, _D))


def kernel(x, edge_attr, senders, receivers, n_atoms,
           eW1, eb1, eW2, eb2, eW3, eb3, eg, ebt,
           nW1, nb1, nW2, nb2, nW3, nb3, ng, nbt):
    ws = _prep_weights(eW1, eb1, eW2, eb2, eW3, eb3, eg, ebt,
                       nW1, nb1, nW2, nb2, nW3, nb3, ng, nbt)
    gath = _sc_gather()
    m = s = None
    for p in range(_P):
        rec_p, snd_p = gath(x,
                            lax.slice(receivers, (p * _EP,), ((p + 1) * _EP,)),
                            lax.slice(senders, (p * _EP,), ((p + 1) * _EP,)))
        m, s = _tc_edge_part(p, rec_p, snd_p, edge_attr, ws, m, s)
    x_out = _node_call(s, ws)
    return (x_out, m)


# speedup vs baseline: 1.1301x; 1.1301x over previous
import functools

import jax
import jax.numpy as jnp
from jax import lax
from jax.experimental import pallas as pl
from jax.experimental.pallas import tpu as pltpu
from jax.experimental.pallas import tpu_sc as plsc

_N = 10000
_K = 16
_D = 128
_DP = _D // 2
_H = 256
_E = _N * _K


_P = 5
_EP = _E // _P
_CH = 128
_NCHUNK = _EP // _CH
_NC = 2
_NS = 16
_NW = _NC * _NS


def _pack_pairs(rows, packed, t):
    for c in range(_D // 16):
        a = rows[2 * t, pl.ds(16 * c, 16)]
        b = rows[2 * t + 1, pl.ds(16 * c, 16)]
        v = plsc.pack(a, b, format=plsc.PackFormat.INTERLEAVED)
        packed[t, pl.ds(16 * c, 16)] = plsc.bitcast(v, jnp.int32)


_NTW = (_NCHUNK + _NW - 1) // _NW


def _sc_gather_kernel(x_hbm, recv_hbm, send_hbm, rec_out, snd_out,
                      ridx, rrows, rpack, sidx, srows, spack, rsem, ssem):
    wid = lax.axis_index("s") * _NC + lax.axis_index("c")

    def chunk_id(t):
        return lax.min(wid + t * _NW, _NCHUNK - 1)

    def start(t, b):
        cid = chunk_id(t)
        base = cid * _CH
        pltpu.sync_copy(recv_hbm.at[pl.ds(base, _CH)], ridx.at[b])
        pltpu.sync_copy(send_hbm.at[pl.ds(base, _CH)], sidx.at[b])
        r1 = pltpu.async_copy(x_hbm.at[ridx.at[b]], rrows.at[b], rsem.at[b])
        r2 = pltpu.async_copy(x_hbm.at[sidx.at[b]], srows.at[b], ssem.at[b])
        return r1, r2

    def finish(t, b, descs):
        cid = chunk_id(t)
        base2 = cid * (_CH // 2)
        descs[0].wait()
        descs[1].wait()

        def pack_body(u, c2):
            _pack_pairs(rrows.at[b], rpack.at[b], u)
            _pack_pairs(srows.at[b], spack.at[b], u)
            return c2

        lax.fori_loop(0, _CH // 2, pack_body, 0)
        pltpu.sync_copy(rpack.at[b], rec_out.at[pl.ds(base2, _CH // 2)])
        pltpu.sync_copy(spack.at[b], snd_out.at[pl.ds(base2, _CH // 2)])

    descs = start(0, 0)
    for t in range(_NTW):
        nxt = None
        if t + 1 < _NTW:
            nxt = start(t + 1, (t + 1) % 2)
        finish(t, t % 2, descs)
        descs = nxt


@functools.cache
def _sc_gather():
    return pl.kernel(
        _sc_gather_kernel,
        mesh=plsc.VectorSubcoreMesh(core_axis_name="c", subcore_axis_name="s"),
        compiler_params=pltpu.CompilerParams(needs_layout_passes=False),
        out_type=(
            jax.ShapeDtypeStruct((_EP // 2, _D), jnp.int32),
            jax.ShapeDtypeStruct((_EP // 2, _D), jnp.int32),
        ),
        scratch_types=[
            pltpu.VMEM((2, _CH), jnp.int32),
            pltpu.VMEM((2, _CH, _D), jnp.float32),
            pltpu.VMEM((2, _CH // 2, _D), jnp.int32),
            pltpu.VMEM((2, _CH), jnp.int32),
            pltpu.VMEM((2, _CH, _D), jnp.float32),
            pltpu.VMEM((2, _CH // 2, _D), jnp.int32),
            pltpu.SemaphoreType.DMA((2,)),
            pltpu.SemaphoreType.DMA((2,)),
        ],
    )


_NB = 400
_BE = _NB * _K
_GRID = _N // _NB
_SPP = _GRID // _P
_NNB = 1000


def _ln(h, g, bt):
    mu = jnp.mean(h, axis=-1, keepdims=True)
    var = jnp.mean((h - mu) * (h - mu), axis=-1, keepdims=True)
    return g * ((h - mu) * lax.rsqrt(var + 1e-5)) + bt


def _unpack_rows(p):
    bf = jnp.bfloat16
    lo = lax.bitcast_convert_type(p << 16, jnp.float32).astype(bf)
    hi = lax.bitcast_convert_type(p & jnp.int32(-65536), jnp.float32).astype(bf)
    return lo, hi


def _tc_kernel(rec, snd, ea,
               w1r, w1s, w1e, b1, w2, b2, w3, b3, g, bt,
               m_out, s_out):
    f32 = jnp.float32
    bf = jnp.bfloat16
    rlo, rhi = _unpack_rows(rec[...])
    slo, shi = _unpack_rows(snd[...])
    h_e = (jnp.dot(rlo, w1r[...], preferred_element_type=f32)
           + jnp.dot(slo, w1s[...], preferred_element_type=f32))
    h_o = (jnp.dot(rhi, w1r[...], preferred_element_type=f32)
           + jnp.dot(shi, w1s[...], preferred_element_type=f32))
    h = jnp.concatenate([h_e[:, None, :], h_o[:, None, :]],
                        axis=1).reshape(_BE, _H)
    h = h + jnp.dot(ea[...].astype(bf), w1e[...], preferred_element_type=f32)
    h = jax.nn.relu(h + b1[...])
    h = jax.nn.relu(jnp.dot(h.astype(bf), w2[...], preferred_element_type=f32)
                    + b2[...])
    m = jnp.dot(h.astype(bf), w3[...], preferred_element_type=f32) + b3[...]
    mln = _ln(m, g[...], bt[...])
    m_out[...] = mln
    grp = (lax.broadcasted_iota(jnp.int32, (_D, 8), 0) // 16
           == lax.broadcasted_iota(jnp.int32, (_D, 8), 1))
    s_out[...] = jnp.dot(mln, grp.astype(f32), preferred_element_type=f32)


def _node_kernel(xin, nw1, nb1, nw2, nb2, nw3, nb3, ng, nbt, x_out):
    f32 = jnp.float32
    bf = jnp.bfloat16
    h = jax.nn.relu(
        jnp.dot(xin[...].astype(bf), nw1[...], preferred_element_type=f32)
        + nb1[...])
    h = jax.nn.relu(jnp.dot(h.astype(bf), nw2[...], preferred_element_type=f32)
                    + nb2[...])
    y = jnp.dot(h.astype(bf), nw3[...], preferred_element_type=f32) + nb3[...]
    x_out[...] = _ln(y, ng[...], nbt[...])


def _const(shape):
    return pl.BlockSpec(shape, lambda i: tuple(0 for _ in shape))


def _tc_edge_part(part, rec_p, snd_p, ea, ws, m_prev, s_prev,
                  interpret=False):
    (w1r, w1s, w1e, b1, w2, b2, w3, b3, g, bt,
     nw1, nb1, nw2, nb2, nw3, nb3, ng, nbt) = ws
    off = part * _SPP
    part_spec = pl.BlockSpec((_BE // 2, _D), lambda i: (i, 0))
    full_spec = pl.BlockSpec((_BE, _D), lambda i: (i + off, 0))
    in_specs = [
        part_spec, part_spec, full_spec,
        _const((_D, _H)), _const((_D, _H)), _const((_D, _H)),
        _const((1, _H)),
        _const((_H, _H)), _const((1, _H)),
        _const((_H, _D)), _const((1, _D)),
        _const((1, _D)), _const((1, _D)),
    ]
    args = [rec_p, snd_p, ea, w1r, w1s, w1e, b1, w2, b2, w3, b3, g, bt]
    aliases = {}
    if m_prev is not None:
        in_specs += [pl.BlockSpec(memory_space=pl.ANY),
                     pl.BlockSpec(memory_space=pl.ANY)]
        args += [m_prev, s_prev]
        aliases = {13: 0, 14: 1}

    def body(*refs):
        _tc_kernel(*refs[:13], refs[-2], refs[-1])

    m, s = pl.pallas_call(
        body,
        grid=(_SPP,),
        in_specs=in_specs,
        out_specs=[
            pl.BlockSpec((_BE, _D), lambda i: (i + off, 0)),
            pl.BlockSpec((_BE, 8), lambda i: (i + off, 0)),
        ],
        out_shape=[
            jax.ShapeDtypeStruct((_E, _D), jnp.float32),
            jax.ShapeDtypeStruct((_E, 8), jnp.float32),
        ],
        input_output_aliases=aliases,
        interpret=interpret,
    )(*args)
    return m, s


def _node_call(s, ws, interpret=False):
    (w1r, w1s, w1e, b1, w2, b2, w3, b3, g, bt,
     nw1, nb1, nw2, nb2, nw3, nb3, ng, nbt) = ws
    xin = s.reshape(_N, _D)
    x_out = pl.pallas_call(
        _node_kernel,
        grid=(_N // _NNB,),
        in_specs=[
            pl.BlockSpec((_NNB, _D), lambda i: (i, 0)),
            _const((_D, _H)), _const((1, _H)),
            _const((_H, _H)), _const((1, _H)),
            _const((_H, _D)), _const((1, _D)),
            _const((1, _D)), _const((1, _D)),
        ],
        out_specs=pl.BlockSpec((_NNB, _D), lambda i: (i, 0)),
        out_shape=jax.ShapeDtypeStruct((_N, _D), jnp.float32),
        interpret=interpret,
    )(xin, nw1, nb1, nw2, nb2, nw3, nb3, ng, nbt)
    return x_out


def _prep_weights(eW1, eb1, eW2, eb2, eW3, eb3, eg, ebt,
                  nW1, nb1, nW2, nb2, nW3, nb3, ng, nbt):
    bf = jnp.bfloat16
    return (eW1[:_D].astype(bf), eW1[_D:2 * _D].astype(bf),
            eW1[2 * _D:].astype(bf),
            eb1.reshape(1, _H), eW2.astype(bf), eb2.reshape(1, _H),
            eW3.astype(bf), eb3.reshape(1, _D),
            eg.reshape(1, _D), ebt.reshape(1, _D),
            nW1.astype(bf), nb1.reshape(1, _H),
            nW2.astype(bf), nb2.reshape(1, _H),
            nW3.astype(bf), nb3.reshape(1, _D),
            ng.reshape(1, _D), nbt.reshape(1, _D))


def kernel(x, edge_attr, senders, receivers, n_atoms,
           eW1, eb1, eW2, eb2, eW3, eb3, eg, ebt,
           nW1, nb1, nW2, nb2, nW3, nb3, ng, nbt):
    ws = _prep_weights(eW1, eb1, eW2, eb2, eW3, eb3, eg, ebt,
                       nW1, nb1, nW2, nb2, nW3, nb3, ng, nbt)
    gath = _sc_gather()
    m = s = None
    for p in range(_P):
        rec_p, snd_p = gath(x,
                            lax.slice(receivers, (p * _EP,), ((p + 1) * _EP,)),
                            lax.slice(senders, (p * _EP,), ((p + 1) * _EP,)))
        m, s = _tc_edge_part(p, rec_p, snd_p, edge_attr, ws, m, s)
    x_out = _node_call(s, ws)
    return (x_out, m)


# speedup vs baseline: 1.1721x; 1.0372x over previous
import functools

import jax
import jax.numpy as jnp
from jax import lax
from jax.experimental import pallas as pl
from jax.experimental.pallas import tpu as pltpu
from jax.experimental.pallas import tpu_sc as plsc

_N = 10000
_K = 16
_D = 128
_DP = _D // 2
_H = 256
_E = _N * _K


_P = 5
_EP = _E // _P
_CH = 128
_NCHUNK = _EP // _CH
_NC = 2
_NS = 16
_NW = _NC * _NS


def _pack_pairs(rows, packed, t):
    for c in range(_D // 16):
        a = rows[2 * t, pl.ds(16 * c, 16)]
        b = rows[2 * t + 1, pl.ds(16 * c, 16)]
        v = plsc.pack(a, b, format=plsc.PackFormat.INTERLEAVED)
        packed[t, pl.ds(16 * c, 16)] = plsc.bitcast(v, jnp.int32)


_NTW = (_NCHUNK + _NW - 1) // _NW


def _sc_gather_kernel(x_hbm, recv_hbm, send_hbm, rec_out, snd_out,
                      ridx, rrows, rpack, sidx, srows, spack,
                      rsem, ssem, risem, sisem, rwsem, swsem):
    wid = lax.axis_index("s") * _NC + lax.axis_index("c")

    def chunk_id(t):
        return lax.min(wid + t * _NW, _NCHUNK - 1)

    def start_idx(t, b):
        base = chunk_id(t) * _CH
        i1 = pltpu.async_copy(recv_hbm.at[pl.ds(base, _CH)], ridx.at[b],
                              risem.at[b])
        i2 = pltpu.async_copy(send_hbm.at[pl.ds(base, _CH)], sidx.at[b],
                              sisem.at[b])
        return i1, i2

    def start_gather(b, idescs):
        idescs[0].wait()
        idescs[1].wait()
        r1 = pltpu.async_copy(x_hbm.at[ridx.at[b]], rrows.at[b], rsem.at[b])
        r2 = pltpu.async_copy(x_hbm.at[sidx.at[b]], srows.at[b], ssem.at[b])
        return r1, r2

    def pack_write(t, b, wdescs):
        base2 = chunk_id(t) * (_CH // 2)
        if wdescs is not None:
            wdescs[0].wait()
            wdescs[1].wait()

        def pack_body(u, c2):
            _pack_pairs(rrows.at[b], rpack.at[b], u)
            _pack_pairs(srows.at[b], spack.at[b], u)
            return c2

        lax.fori_loop(0, _CH // 2, pack_body, 0, unroll=4)
        w1 = pltpu.async_copy(rpack.at[b], rec_out.at[pl.ds(base2, _CH // 2)],
                              rwsem.at[b])
        w2 = pltpu.async_copy(spack.at[b], snd_out.at[pl.ds(base2, _CH // 2)],
                              swsem.at[b])
        return w1, w2

    idescs = [start_idx(0, 0), start_idx(1, 1)]
    gdescs = [start_gather(0, idescs[0]), None]
    wdescs = [None, None]
    for t in range(_NTW):
        b = t % 2
        nb = (t + 1) % 2
        if t + 1 < _NTW:
            gdescs[nb] = start_gather(nb, idescs[nb])
        gdescs[b][0].wait()
        gdescs[b][1].wait()
        if t + 2 < _NTW:
            idescs[b] = start_idx(t + 2, b)
        wdescs[b] = pack_write(t, b, wdescs[b])
    wdescs[0][0].wait()
    wdescs[0][1].wait()
    wdescs[1][0].wait()
    wdescs[1][1].wait()


@functools.cache
def _sc_gather():
    return pl.kernel(
        _sc_gather_kernel,
        mesh=plsc.VectorSubcoreMesh(core_axis_name="c", subcore_axis_name="s"),
        compiler_params=pltpu.CompilerParams(needs_layout_passes=False),
        out_type=(
            jax.ShapeDtypeStruct((_EP // 2, _D), jnp.int32),
            jax.ShapeDtypeStruct((_EP // 2, _D), jnp.int32),
        ),
        scratch_types=[
            pltpu.VMEM((2, _CH), jnp.int32),
            pltpu.VMEM((2, _CH, _D), jnp.float32),
            pltpu.VMEM((2, _CH // 2, _D), jnp.int32),
            pltpu.VMEM((2, _CH), jnp.int32),
            pltpu.VMEM((2, _CH, _D), jnp.float32),
            pltpu.VMEM((2, _CH // 2, _D), jnp.int32),
            pltpu.SemaphoreType.DMA((2,)),
            pltpu.SemaphoreType.DMA((2,)),
            pltpu.SemaphoreType.DMA((2,)),
            pltpu.SemaphoreType.DMA((2,)),
            pltpu.SemaphoreType.DMA((2,)),
            pltpu.SemaphoreType.DMA((2,)),
        ],
    )


_NB = 400
_BE = _NB * _K
_GRID = _N // _NB
_SPP = _GRID // _P
_NNB = 1000


def _ln(h, g, bt):
    mu = jnp.mean(h, axis=-1, keepdims=True)
    var = jnp.mean((h - mu) * (h - mu), axis=-1, keepdims=True)
    return g * ((h - mu) * lax.rsqrt(var + 1e-5)) + bt


def _unpack_rows(p):
    bf = jnp.bfloat16
    lo = lax.bitcast_convert_type(p << 16, jnp.float32).astype(bf)
    hi = lax.bitcast_convert_type(p & jnp.int32(-65536), jnp.float32).astype(bf)
    return lo, hi


def _tc_kernel(rec, snd, ea,
               w1r, w1s, w1e, b1, w2, b2, w3, b3, g, bt,
               m_out, s_out):
    f32 = jnp.float32
    bf = jnp.bfloat16
    rlo, rhi = _unpack_rows(rec[...])
    slo, shi = _unpack_rows(snd[...])
    h_e = (jnp.dot(rlo, w1r[...], preferred_element_type=f32)
           + jnp.dot(slo, w1s[...], preferred_element_type=f32))
    h_o = (jnp.dot(rhi, w1r[...], preferred_element_type=f32)
           + jnp.dot(shi, w1s[...], preferred_element_type=f32))
    h = jnp.concatenate([h_e[:, None, :], h_o[:, None, :]],
                        axis=1).reshape(_BE, _H)
    h = h + jnp.dot(ea[...].astype(bf), w1e[...], preferred_element_type=f32)
    h = jax.nn.relu(h + b1[...])
    h = jax.nn.relu(jnp.dot(h.astype(bf), w2[...], preferred_element_type=f32)
                    + b2[...])
    m = jnp.dot(h.astype(bf), w3[...], preferred_element_type=f32) + b3[...]
    mln = _ln(m, g[...], bt[...])
    m_out[...] = mln
    grp = (lax.broadcasted_iota(jnp.int32, (_D, 8), 0) // 16
           == lax.broadcasted_iota(jnp.int32, (_D, 8), 1))
    s_out[...] = jnp.dot(mln, grp.astype(f32), preferred_element_type=f32)


def _node_kernel(xin, nw1, nb1, nw2, nb2, nw3, nb3, ng, nbt, x_out):
    f32 = jnp.float32
    bf = jnp.bfloat16
    h = jax.nn.relu(
        jnp.dot(xin[...].astype(bf), nw1[...], preferred_element_type=f32)
        + nb1[...])
    h = jax.nn.relu(jnp.dot(h.astype(bf), nw2[...], preferred_element_type=f32)
                    + nb2[...])
    y = jnp.dot(h.astype(bf), nw3[...], preferred_element_type=f32) + nb3[...]
    x_out[...] = _ln(y, ng[...], nbt[...])


def _const(shape):
    return pl.BlockSpec(shape, lambda i: tuple(0 for _ in shape))


def _tc_edge_part(part, rec_p, snd_p, ea, ws, m_prev, s_prev,
                  interpret=False):
    (w1r, w1s, w1e, b1, w2, b2, w3, b3, g, bt,
     nw1, nb1, nw2, nb2, nw3, nb3, ng, nbt) = ws
    off = part * _SPP
    part_spec = pl.BlockSpec((_BE // 2, _D), lambda i: (i, 0))
    full_spec = pl.BlockSpec((_BE, _D), lambda i: (i + off, 0))
    in_specs = [
        part_spec, part_spec, full_spec,
        _const((_D, _H)), _const((_D, _H)), _const((_D, _H)),
        _const((1, _H)),
        _const((_H, _H)), _const((1, _H)),
        _const((_H, _D)), _const((1, _D)),
        _const((1, _D)), _const((1, _D)),
    ]
    args = [rec_p, snd_p, ea, w1r, w1s, w1e, b1, w2, b2, w3, b3, g, bt]
    aliases = {}
    if m_prev is not None:
        in_specs += [pl.BlockSpec(memory_space=pl.ANY),
                     pl.BlockSpec(memory_space=pl.ANY)]
        args += [m_prev, s_prev]
        aliases = {13: 0, 14: 1}

    def body(*refs):
        _tc_kernel(*refs[:13], refs[-2], refs[-1])

    m, s = pl.pallas_call(
        body,
        grid=(_SPP,),
        in_specs=in_specs,
        out_specs=[
            pl.BlockSpec((_BE, _D), lambda i: (i + off, 0)),
            pl.BlockSpec((_BE, 8), lambda i: (i + off, 0)),
        ],
        out_shape=[
            jax.ShapeDtypeStruct((_E, _D), jnp.float32),
            jax.ShapeDtypeStruct((_E, 8), jnp.float32),
        ],
        input_output_aliases=aliases,
        interpret=interpret,
    )(*args)
    return m, s


def _node_call(s, ws, interpret=False):
    (w1r, w1s, w1e, b1, w2, b2, w3, b3, g, bt,
     nw1, nb1, nw2, nb2, nw3, nb3, ng, nbt) = ws
    xin = s.reshape(_N, _D)
    x_out = pl.pallas_call(
        _node_kernel,
        grid=(_N // _NNB,),
        in_specs=[
            pl.BlockSpec((_NNB, _D), lambda i: (i, 0)),
            _const((_D, _H)), _const((1, _H)),
            _const((_H, _H)), _const((1, _H)),
            _const((_H, _D)), _const((1, _D)),
            _const((1, _D)), _const((1, _D)),
        ],
        out_specs=pl.BlockSpec((_NNB, _D), lambda i: (i, 0)),
        out_shape=jax.ShapeDtypeStruct((_N, _D), jnp.float32),
        interpret=interpret,
    )(xin, nw1, nb1, nw2, nb2, nw3, nb3, ng, nbt)
    return x_out


def _prep_weights(eW1, eb1, eW2, eb2, eW3, eb3, eg, ebt,
                  nW1, nb1, nW2, nb2, nW3, nb3, ng, nbt):
    bf = jnp.bfloat16
    return (eW1[:_D].astype(bf), eW1[_D:2 * _D].astype(bf),
            eW1[2 * _D:].astype(bf),
            eb1.reshape(1, _H), eW2.astype(bf), eb2.reshape(1, _H),
            eW3.astype(bf), eb3.reshape(1, _D),
            eg.reshape(1, _D), ebt.reshape(1, _D),
            nW1.astype(bf), nb1.reshape(1, _H),
            nW2.astype(bf), nb2.reshape(1, _H),
            nW3.astype(bf), nb3.reshape(1, _D),
            ng.reshape(1, _D), nbt.reshape(1, _D))


def kernel(x, edge_attr, senders, receivers, n_atoms,
           eW1, eb1, eW2, eb2, eW3, eb3, eg, ebt,
           nW1, nb1, nW2, nb2, nW3, nb3, ng, nbt):
    ws = _prep_weights(eW1, eb1, eW2, eb2, eW3, eb3, eg, ebt,
                       nW1, nb1, nW2, nb2, nW3, nb3, ng, nbt)
    gath = _sc_gather()
    m = s = None
    for p in range(_P):
        rec_p, snd_p = gath(x,
                            lax.slice(receivers, (p * _EP,), ((p + 1) * _EP,)),
                            lax.slice(senders, (p * _EP,), ((p + 1) * _EP,)))
        m, s = _tc_edge_part(p, rec_p, snd_p, edge_attr, ws, m, s)
    x_out = _node_call(s, ws)
    return (x_out, m)


# speedup vs baseline: 1.3340x; 1.1381x over previous
import functools

import jax
import jax.numpy as jnp
from jax import lax
from jax.experimental import pallas as pl
from jax.experimental.pallas import tpu as pltpu
from jax.experimental.pallas import tpu_sc as plsc

_N = 10000
_K = 16
_D = 128
_DP = _D // 2
_H = 256
_E = _N * _K


_P = 5
_EP = _E // _P
_CH = 128
_NCHUNK = _EP // _CH
_NC = 2
_NS = 16
_NW = _NC * _NS


_NTW = (_NCHUNK + _NW - 1) // _NW


def _sc_gather_kernel(x_hbm, recv_hbm, send_hbm, rec_out, snd_out,
                      ridx, rrows, sidx, srows,
                      rsem, ssem, risem, sisem, rwsem, swsem):
    wid = lax.axis_index("s") * _NC + lax.axis_index("c")

    def chunk_id(t):
        return lax.min(wid + t * _NW, _NCHUNK - 1)

    def start_idx(t, b):
        base = chunk_id(t) * _CH
        i1 = pltpu.async_copy(recv_hbm.at[pl.ds(base, _CH)], ridx.at[b],
                              risem.at[b])
        i2 = pltpu.async_copy(send_hbm.at[pl.ds(base, _CH)], sidx.at[b],
                              sisem.at[b])
        return i1, i2

    def start_gather(b, idescs):
        idescs[0].wait()
        idescs[1].wait()
        r1 = pltpu.async_copy(x_hbm.at[ridx.at[b]], rrows.at[b], rsem.at[b])
        r2 = pltpu.async_copy(x_hbm.at[sidx.at[b]], srows.at[b], ssem.at[b])
        return r1, r2

    def write_out(t, b):
        base = chunk_id(t) * _CH
        w1 = pltpu.async_copy(rrows.at[b], rec_out.at[pl.ds(base, _CH)],
                              rwsem.at[b])
        w2 = pltpu.async_copy(srows.at[b], snd_out.at[pl.ds(base, _CH)],
                              swsem.at[b])
        return w1, w2

    idescs = [start_idx(0, 0), start_idx(1, 1)]
    gdescs = [start_gather(0, idescs[0]), None]
    wdescs = [None, None]
    for t in range(_NTW):
        b = t % 2
        nb = (t + 1) % 2
        if t + 1 < _NTW:
            if wdescs[nb] is not None:
                wdescs[nb][0].wait()
                wdescs[nb][1].wait()
                wdescs[nb] = None
            gdescs[nb] = start_gather(nb, idescs[nb])
        gdescs[b][0].wait()
        gdescs[b][1].wait()
        if t + 2 < _NTW:
            idescs[b] = start_idx(t + 2, b)
        wdescs[b] = write_out(t, b)
    for wd in wdescs:
        if wd is not None:
            wd[0].wait()
            wd[1].wait()


@functools.cache
def _sc_gather():
    return pl.kernel(
        _sc_gather_kernel,
        mesh=plsc.VectorSubcoreMesh(core_axis_name="c", subcore_axis_name="s"),
        compiler_params=pltpu.CompilerParams(needs_layout_passes=False),
        out_type=(
            jax.ShapeDtypeStruct((_EP, _D), jnp.float32),
            jax.ShapeDtypeStruct((_EP, _D), jnp.float32),
        ),
        scratch_types=[
            pltpu.VMEM((2, _CH), jnp.int32),
            pltpu.VMEM((2, _CH, _D), jnp.float32),
            pltpu.VMEM((2, _CH), jnp.int32),
            pltpu.VMEM((2, _CH, _D), jnp.float32),
            pltpu.SemaphoreType.DMA((2,)),
            pltpu.SemaphoreType.DMA((2,)),
            pltpu.SemaphoreType.DMA((2,)),
            pltpu.SemaphoreType.DMA((2,)),
            pltpu.SemaphoreType.DMA((2,)),
            pltpu.SemaphoreType.DMA((2,)),
        ],
    )


_NB = 400
_BE = _NB * _K
_GRID = _N // _NB
_SPP = _GRID // _P
_NNB = 1000


def _ln(h, g, bt):
    mu = jnp.mean(h, axis=-1, keepdims=True)
    var = jnp.mean((h - mu) * (h - mu), axis=-1, keepdims=True)
    return g * ((h - mu) * lax.rsqrt(var + 1e-5)) + bt


def _tc_kernel(rec, snd, ea,
               w1r, w1s, w1e, b1, w2, b2, w3, b3, g, bt,
               m_out, s_out):
    f32 = jnp.float32
    bf = jnp.bfloat16
    h = jnp.dot(rec[...].astype(bf), w1r[...], preferred_element_type=f32)
    h = h + jnp.dot(snd[...].astype(bf), w1s[...], preferred_element_type=f32)
    h = h + jnp.dot(ea[...].astype(bf), w1e[...], preferred_element_type=f32)
    h = jax.nn.relu(h + b1[...])
    h = jax.nn.relu(jnp.dot(h.astype(bf), w2[...], preferred_element_type=f32)
                    + b2[...])
    m = jnp.dot(h.astype(bf), w3[...], preferred_element_type=f32) + b3[...]
    mln = _ln(m, g[...], bt[...])
    m_out[...] = mln
    grp = (lax.broadcasted_iota(jnp.int32, (_D, 8), 0) // 16
           == lax.broadcasted_iota(jnp.int32, (_D, 8), 1))
    s_out[...] = jnp.dot(mln, grp.astype(f32), preferred_element_type=f32)


def _node_kernel(xin, nw1, nb1, nw2, nb2, nw3, nb3, ng, nbt, x_out):
    f32 = jnp.float32
    bf = jnp.bfloat16
    h = jax.nn.relu(
        jnp.dot(xin[...].astype(bf), nw1[...], preferred_element_type=f32)
        + nb1[...])
    h = jax.nn.relu(jnp.dot(h.astype(bf), nw2[...], preferred_element_type=f32)
                    + nb2[...])
    y = jnp.dot(h.astype(bf), nw3[...], preferred_element_type=f32) + nb3[...]
    x_out[...] = _ln(y, ng[...], nbt[...])


def _const(shape):
    return pl.BlockSpec(shape, lambda i: tuple(0 for _ in shape))


def _tc_edge_part(part, rec_p, snd_p, ea, ws, m_prev, s_prev,
                  interpret=False):
    (w1r, w1s, w1e, b1, w2, b2, w3, b3, g, bt,
     nw1, nb1, nw2, nb2, nw3, nb3, ng, nbt) = ws
    off = part * _SPP
    part_spec = pl.BlockSpec((_BE, _D), lambda i: (i, 0))
    full_spec = pl.BlockSpec((_BE, _D), lambda i: (i + off, 0))
    in_specs = [
        part_spec, part_spec, full_spec,
        _const((_D, _H)), _const((_D, _H)), _const((_D, _H)),
        _const((1, _H)),
        _const((_H, _H)), _const((1, _H)),
        _const((_H, _D)), _const((1, _D)),
        _const((1, _D)), _const((1, _D)),
    ]
    args = [rec_p, snd_p, ea, w1r, w1s, w1e, b1, w2, b2, w3, b3, g, bt]
    aliases = {}
    if m_prev is not None:
        in_specs += [pl.BlockSpec(memory_space=pl.ANY),
                     pl.BlockSpec(memory_space=pl.ANY)]
        args += [m_prev, s_prev]
        aliases = {13: 0, 14: 1}

    def body(*refs):
        _tc_kernel(*refs[:13], refs[-2], refs[-1])

    m, s = pl.pallas_call(
        body,
        grid=(_SPP,),
        in_specs=in_specs,
        out_specs=[
            pl.BlockSpec((_BE, _D), lambda i: (i + off, 0)),
            pl.BlockSpec((_BE, 8), lambda i: (i + off, 0)),
        ],
        out_shape=[
            jax.ShapeDtypeStruct((_E, _D), jnp.float32),
            jax.ShapeDtypeStruct((_E, 8), jnp.float32),
        ],
        input_output_aliases=aliases,
        interpret=interpret,
    )(*args)
    return m, s


def _node_call(s, ws, interpret=False):
    (w1r, w1s, w1e, b1, w2, b2, w3, b3, g, bt,
     nw1, nb1, nw2, nb2, nw3, nb3, ng, nbt) = ws
    xin = s.reshape(_N, _D)
    x_out = pl.pallas_call(
        _node_kernel,
        grid=(_N // _NNB,),
        in_specs=[
            pl.BlockSpec((_NNB, _D), lambda i: (i, 0)),
            _const((_D, _H)), _const((1, _H)),
            _const((_H, _H)), _const((1, _H)),
            _const((_H, _D)), _const((1, _D)),
            _const((1, _D)), _const((1, _D)),
        ],
        out_specs=pl.BlockSpec((_NNB, _D), lambda i: (i, 0)),
        out_shape=jax.ShapeDtypeStruct((_N, _D), jnp.float32),
        interpret=interpret,
    )(xin, nw1, nb1, nw2, nb2, nw3, nb3, ng, nbt)
    return x_out


def _prep_weights(eW1, eb1, eW2, eb2, eW3, eb3, eg, ebt,
                  nW1, nb1, nW2, nb2, nW3, nb3, ng, nbt):
    bf = jnp.bfloat16
    return (eW1[:_D].astype(bf), eW1[_D:2 * _D].astype(bf),
            eW1[2 * _D:].astype(bf),
            eb1.reshape(1, _H), eW2.astype(bf), eb2.reshape(1, _H),
            eW3.astype(bf), eb3.reshape(1, _D),
            eg.reshape(1, _D), ebt.reshape(1, _D),
            nW1.astype(bf), nb1.reshape(1, _H),
            nW2.astype(bf), nb2.reshape(1, _H),
            nW3.astype(bf), nb3.reshape(1, _D),
            ng.reshape(1, _D), nbt.reshape(1, _D))


def kernel(x, edge_attr, senders, receivers, n_atoms,
           eW1, eb1, eW2, eb2, eW3, eb3, eg, ebt,
           nW1, nb1, nW2, nb2, nW3, nb3, ng, nbt):
    ws = _prep_weights(eW1, eb1, eW2, eb2, eW3, eb3, eg, ebt,
                       nW1, nb1, nW2, nb2, nW3, nb3, ng, nbt)
    gath = _sc_gather()
    m = s = None
    for p in range(_P):
        rec_p, snd_p = gath(x,
                            lax.slice(receivers, (p * _EP,), ((p + 1) * _EP,)),
                            lax.slice(senders, (p * _EP,), ((p + 1) * _EP,)))
        m, s = _tc_edge_part(p, rec_p, snd_p, edge_attr, ws, m, s)
    x_out = _node_call(s, ws)
    return (x_out, m)


# speedup vs baseline: 1.3656x; 1.0237x over previous
import functools

import jax
import jax.numpy as jnp
from jax import lax
from jax.experimental import pallas as pl
from jax.experimental.pallas import tpu as pltpu
from jax.experimental.pallas import tpu_sc as plsc

_N = 10000
_K = 16
_D = 128
_DP = _D // 2
_H = 256
_E = _N * _K


_P = 5
_EP = _E // _P
_CH = 128
_NCHUNK = _EP // _CH
_NC = 2
_NS = 16
_NW = _NC * _NS


def _sc_gather_kernel(x_hbm, recv_hbm, send_hbm, rec_out, snd_out,
                      ridx, rrows, sidx, srows, rsem, ssem):
    wid = lax.axis_index("s") * _NC + lax.axis_index("c")
    nt = (_NCHUNK - wid + _NW - 1) // _NW

    def body(t, carry):
        base = (wid + t * _NW) * _CH
        pltpu.sync_copy(recv_hbm.at[pl.ds(base, _CH)], ridx)
        pltpu.sync_copy(send_hbm.at[pl.ds(base, _CH)], sidx)
        r1 = pltpu.async_copy(x_hbm.at[ridx], rrows, rsem)
        r2 = pltpu.async_copy(x_hbm.at[sidx], srows, ssem)
        r1.wait()
        r2.wait()
        pltpu.sync_copy(rrows, rec_out.at[pl.ds(base, _CH)])
        pltpu.sync_copy(srows, snd_out.at[pl.ds(base, _CH)])
        return carry

    lax.fori_loop(0, nt, body, 0)


@functools.cache
def _sc_gather():
    return pl.kernel(
        _sc_gather_kernel,
        mesh=plsc.VectorSubcoreMesh(core_axis_name="c", subcore_axis_name="s"),
        out_type=(
            jax.ShapeDtypeStruct((_EP, _D), jnp.float32),
            jax.ShapeDtypeStruct((_EP, _D), jnp.float32),
        ),
        scratch_types=[
            pltpu.VMEM((_CH,), jnp.int32),
            pltpu.VMEM((_CH, _D), jnp.float32),
            pltpu.VMEM((_CH,), jnp.int32),
            pltpu.VMEM((_CH, _D), jnp.float32),
            pltpu.SemaphoreType.DMA,
            pltpu.SemaphoreType.DMA,
        ],
    )


_NB = 500
_BE = _NB * _K
_GRID = _N // _NB
_SPP = _GRID // _P
_NNB = 1000


def _ln(h, g, bt):
    mu = jnp.mean(h, axis=-1, keepdims=True)
    var = jnp.mean((h - mu) * (h - mu), axis=-1, keepdims=True)
    return g * ((h - mu) * lax.rsqrt(var + 1e-5)) + bt


def _tc_kernel(rec, snd, ea,
               w1r, w1s, w1e, b1, w2, b2, w3, b3, g, bt,
               m_out, s_out):
    f32 = jnp.float32
    bf = jnp.bfloat16
    h = jnp.dot(rec[...].astype(bf), w1r[...], preferred_element_type=f32)
    h = h + jnp.dot(snd[...].astype(bf), w1s[...], preferred_element_type=f32)
    h = h + jnp.dot(ea[...].astype(bf), w1e[...], preferred_element_type=f32)
    h = jax.nn.relu(h + b1[...])
    h = jax.nn.relu(jnp.dot(h.astype(bf), w2[...], preferred_element_type=f32)
                    + b2[...])
    m = jnp.dot(h.astype(bf), w3[...], preferred_element_type=f32) + b3[...]
    mln = _ln(m, g[...], bt[...])
    m_out[...] = mln
    grp = (lax.broadcasted_iota(jnp.int32, (_D, 8), 0) // 16
           == lax.broadcasted_iota(jnp.int32, (_D, 8), 1))
    s_out[...] = jnp.dot(mln, grp.astype(f32), preferred_element_type=f32)


def _node_kernel(xin, nw1, nb1, nw2, nb2, nw3, nb3, ng, nbt, x_out):
    f32 = jnp.float32
    bf = jnp.bfloat16
    h = jax.nn.relu(
        jnp.dot(xin[...].astype(bf), nw1[...], preferred_element_type=f32)
        + nb1[...])
    h = jax.nn.relu(jnp.dot(h.astype(bf), nw2[...], preferred_element_type=f32)
                    + nb2[...])
    y = jnp.dot(h.astype(bf), nw3[...], preferred_element_type=f32) + nb3[...]
    x_out[...] = _ln(y, ng[...], nbt[...])


def _const(shape):
    return pl.BlockSpec(shape, lambda i: tuple(0 for _ in shape))


def _tc_edge_part(part, rec_p, snd_p, ea, ws, m_prev, s_prev,
                  interpret=False):
    (w1r, w1s, w1e, b1, w2, b2, w3, b3, g, bt,
     nw1, nb1, nw2, nb2, nw3, nb3, ng, nbt) = ws
    off = part * _SPP
    part_spec = pl.BlockSpec((_BE, _D), lambda i: (i, 0))
    full_spec = pl.BlockSpec((_BE, _D), lambda i: (i + off, 0))
    in_specs = [
        part_spec, part_spec, full_spec,
        _const((_D, _H)), _const((_D, _H)), _const((_D, _H)),
        _const((1, _H)),
        _const((_H, _H)), _const((1, _H)),
        _const((_H, _D)), _const((1, _D)),
        _const((1, _D)), _const((1, _D)),
    ]
    args = [rec_p, snd_p, ea, w1r, w1s, w1e, b1, w2, b2, w3, b3, g, bt]
    aliases = {}
    if m_prev is not None:
        in_specs += [pl.BlockSpec(memory_space=pl.ANY),
                     pl.BlockSpec(memory_space=pl.ANY)]
        args += [m_prev, s_prev]
        aliases = {13: 0, 14: 1}

    def body(*refs):
        _tc_kernel(*refs[:13], refs[-2], refs[-1])

    m, s = pl.pallas_call(
        body,
        grid=(_SPP,),
        in_specs=in_specs,
        out_specs=[
            pl.BlockSpec((_BE, _D), lambda i: (i + off, 0)),
            pl.BlockSpec((_BE, 8), lambda i: (i + off, 0)),
        ],
        out_shape=[
            jax.ShapeDtypeStruct((_E, _D), jnp.float32),
            jax.ShapeDtypeStruct((_E, 8), jnp.float32),
        ],
        input_output_aliases=aliases,
        interpret=interpret,
    )(*args)
    return m, s


def _node_call(s, ws, interpret=False):
    (w1r, w1s, w1e, b1, w2, b2, w3, b3, g, bt,
     nw1, nb1, nw2, nb2, nw3, nb3, ng, nbt) = ws
    xin = s.reshape(_N, _D)
    x_out = pl.pallas_call(
        _node_kernel,
        grid=(_N // _NNB,),
        in_specs=[
            pl.BlockSpec((_NNB, _D), lambda i: (i, 0)),
            _const((_D, _H)), _const((1, _H)),
            _const((_H, _H)), _const((1, _H)),
            _const((_H, _D)), _const((1, _D)),
            _const((1, _D)), _const((1, _D)),
        ],
        out_specs=pl.BlockSpec((_NNB, _D), lambda i: (i, 0)),
        out_shape=jax.ShapeDtypeStruct((_N, _D), jnp.float32),
        interpret=interpret,
    )(xin, nw1, nb1, nw2, nb2, nw3, nb3, ng, nbt)
    return x_out


def _prep_weights(eW1, eb1, eW2, eb2, eW3, eb3, eg, ebt,
                  nW1, nb1, nW2, nb2, nW3, nb3, ng, nbt):
    bf = jnp.bfloat16
    return (eW1[:_D].astype(bf), eW1[_D:2 * _D].astype(bf),
            eW1[2 * _D:].astype(bf),
            eb1.reshape(1, _H), eW2.astype(bf), eb2.reshape(1, _H),
            eW3.astype(bf), eb3.reshape(1, _D),
            eg.reshape(1, _D), ebt.reshape(1, _D),
            nW1.astype(bf), nb1.reshape(1, _H),
            nW2.astype(bf), nb2.reshape(1, _H),
            nW3.astype(bf), nb3.reshape(1, _D),
            ng.reshape(1, _D), nbt.reshape(1, _D))


def kernel(x, edge_attr, senders, receivers, n_atoms,
           eW1, eb1, eW2, eb2, eW3, eb3, eg, ebt,
           nW1, nb1, nW2, nb2, nW3, nb3, ng, nbt):
    ws = _prep_weights(eW1, eb1, eW2, eb2, eW3, eb3, eg, ebt,
                       nW1, nb1, nW2, nb2, nW3, nb3, ng, nbt)
    gath = _sc_gather()
    m = s = None
    for p in range(_P):
        rec_p, snd_p = gath(x,
                            lax.slice(receivers, (p * _EP,), ((p + 1) * _EP,)),
                            lax.slice(senders, (p * _EP,), ((p + 1) * _EP,)))
        m, s = _tc_edge_part(p, rec_p, snd_p, edge_attr, ws, m, s)
    x_out = _node_call(s, ws)
    return (x_out, m)


# speedup vs baseline: 1.3664x; 1.0006x over previous
import functools

import jax
import jax.numpy as jnp
from jax import lax
from jax.experimental import pallas as pl
from jax.experimental.pallas import tpu as pltpu
from jax.experimental.pallas import tpu_sc as plsc

_N = 10000
_K = 16
_D = 128
_DP = _D // 2
_H = 256
_E = _N * _K


_P = 5
_EP = _E // _P
_CH = 128
_NCHUNK = _EP // _CH
_NC = 2
_NS = 16
_NW = _NC * _NS


def _sc_gather_kernel(x_hbm, recv_hbm, send_hbm, rec_out, snd_out,
                      ridx, rrows, sidx, srows, rsem, ssem):
    wid = lax.axis_index("s") * _NC + lax.axis_index("c")
    nt = (_NCHUNK - wid + _NW - 1) // _NW

    def body(t, carry):
        base = (wid + t * _NW) * _CH
        pltpu.sync_copy(recv_hbm.at[pl.ds(base, _CH)], ridx)
        pltpu.sync_copy(send_hbm.at[pl.ds(base, _CH)], sidx)
        r1 = pltpu.async_copy(x_hbm.at[ridx], rrows, rsem)
        r2 = pltpu.async_copy(x_hbm.at[sidx], srows, ssem)
        r1.wait()
        r2.wait()
        pltpu.sync_copy(rrows, rec_out.at[pl.ds(base, _CH)])
        pltpu.sync_copy(srows, snd_out.at[pl.ds(base, _CH)])
        return carry

    lax.fori_loop(0, nt, body, 0)


@functools.cache
def _sc_gather():
    return pl.kernel(
        _sc_gather_kernel,
        mesh=plsc.VectorSubcoreMesh(core_axis_name="c", subcore_axis_name="s"),
        out_type=(
            jax.ShapeDtypeStruct((_EP, _D), jnp.float32),
            jax.ShapeDtypeStruct((_EP, _D), jnp.float32),
        ),
        scratch_types=[
            pltpu.VMEM((_CH,), jnp.int32),
            pltpu.VMEM((_CH, _D), jnp.float32),
            pltpu.VMEM((_CH,), jnp.int32),
            pltpu.VMEM((_CH, _D), jnp.float32),
            pltpu.SemaphoreType.DMA,
            pltpu.SemaphoreType.DMA,
        ],
    )


_NB = 500
_BE = _NB * _K
_GRID = _N // _NB
_SPP = _GRID // _P
_NNB = 1000


def _ln(h, g, bt):
    mu = jnp.mean(h, axis=-1, keepdims=True)
    var = jnp.mean((h - mu) * (h - mu), axis=-1, keepdims=True)
    return g * ((h - mu) * lax.rsqrt(var + 1e-5)) + bt


def _tc_kernel(rec, snd, ea,
               w1r, w1s, w1e, b1, w2, b2, w3, b3, g, bt,
               m_out, s_out):
    f32 = jnp.float32
    bf = jnp.bfloat16
    h = jnp.dot(rec[...].astype(bf), w1r[...], preferred_element_type=f32)
    h = h + jnp.dot(snd[...].astype(bf), w1s[...], preferred_element_type=f32)
    h = h + jnp.dot(ea[...].astype(bf), w1e[...], preferred_element_type=f32)
    h = jax.nn.relu(h + b1[...])
    h = jax.nn.relu(jnp.dot(h.astype(bf), w2[...], preferred_element_type=f32)
                    + b2[...])
    m = jnp.dot(h.astype(bf), w3[...], preferred_element_type=f32) + b3[...]
    mln = _ln(m, g[...], bt[...])
    m_out[...] = mln
    grp = (lax.broadcasted_iota(jnp.int32, (_D, 8), 0) // 16
           == lax.broadcasted_iota(jnp.int32, (_D, 8), 1))
    s_out[...] = jnp.dot(mln, grp.astype(f32), preferred_element_type=f32)


def _node_kernel(xin, nw1, nb1, nw2, nb2, nw3, nb3, ng, nbt, x_out):
    f32 = jnp.float32
    bf = jnp.bfloat16
    h = jax.nn.relu(
        jnp.dot(xin[...].astype(bf), nw1[...], preferred_element_type=f32)
        + nb1[...])
    h = jax.nn.relu(jnp.dot(h.astype(bf), nw2[...], preferred_element_type=f32)
                    + nb2[...])
    y = jnp.dot(h.astype(bf), nw3[...], preferred_element_type=f32) + nb3[...]
    x_out[...] = _ln(y, ng[...], nbt[...])


def _const(shape):
    return pl.BlockSpec(shape, lambda i: tuple(0 for _ in shape))


def _tc_edge_part(part, rec_p, snd_p, ea, ws, m_prev, s_prev,
                  interpret=False):
    (w1r, w1s, w1e, b1, w2, b2, w3, b3, g, bt,
     nw1, nb1, nw2, nb2, nw3, nb3, ng, nbt) = ws
    off = part * _SPP
    part_spec = pl.BlockSpec((_BE, _D), lambda i: (i, 0))
    full_spec = pl.BlockSpec((_BE, _D), lambda i: (i + off, 0))
    in_specs = [
        part_spec, part_spec, full_spec,
        _const((_D, _H)), _const((_D, _H)), _const((_D, _H)),
        _const((1, _H)),
        _const((_H, _H)), _const((1, _H)),
        _const((_H, _D)), _const((1, _D)),
        _const((1, _D)), _const((1, _D)),
    ]
    args = [rec_p, snd_p, ea, w1r, w1s, w1e, b1, w2, b2, w3, b3, g, bt]
    aliases = {}
    if m_prev is not None:
        in_specs += [pl.BlockSpec(memory_space=pl.ANY),
                     pl.BlockSpec(memory_space=pl.ANY)]
        args += [m_prev, s_prev]
        aliases = {13: 0, 14: 1}

    def body(*refs):
        _tc_kernel(*refs[:13], refs[-2], refs[-1])

    m, s = pl.pallas_call(
        body,
        grid=(_SPP,),
        in_specs=in_specs,
        out_specs=[
            pl.BlockSpec((_BE, _D), lambda i: (i + off, 0)),
            pl.BlockSpec((_BE, 8), lambda i: (i + off, 0)),
        ],
        out_shape=[
            jax.ShapeDtypeStruct((_E, _D), jnp.float32),
            jax.ShapeDtypeStruct((_E, 8), jnp.float32),
        ],
        input_output_aliases=aliases,
        compiler_params=pltpu.CompilerParams(
            vmem_limit_bytes=100 * 1024 * 1024),
        interpret=interpret,
    )(*args)
    return m, s


def _node_call(s, ws, interpret=False):
    (w1r, w1s, w1e, b1, w2, b2, w3, b3, g, bt,
     nw1, nb1, nw2, nb2, nw3, nb3, ng, nbt) = ws
    xin = s.reshape(_N, _D)
    x_out = pl.pallas_call(
        _node_kernel,
        grid=(_N // _NNB,),
        in_specs=[
            pl.BlockSpec((_NNB, _D), lambda i: (i, 0)),
            _const((_D, _H)), _const((1, _H)),
            _const((_H, _H)), _const((1, _H)),
            _const((_H, _D)), _const((1, _D)),
            _const((1, _D)), _const((1, _D)),
        ],
        out_specs=pl.BlockSpec((_NNB, _D), lambda i: (i, 0)),
        out_shape=jax.ShapeDtypeStruct((_N, _D), jnp.float32),
        interpret=interpret,
    )(xin, nw1, nb1, nw2, nb2, nw3, nb3, ng, nbt)
    return x_out


def _prep_weights(eW1, eb1, eW2, eb2, eW3, eb3, eg, ebt,
                  nW1, nb1, nW2, nb2, nW3, nb3, ng, nbt):
    bf = jnp.bfloat16
    return (eW1[:_D].astype(bf), eW1[_D:2 * _D].astype(bf),
            eW1[2 * _D:].astype(bf),
            eb1.reshape(1, _H), eW2.astype(bf), eb2.reshape(1, _H),
            eW3.astype(bf), eb3.reshape(1, _D),
            eg.reshape(1, _D), ebt.reshape(1, _D),
            nW1.astype(bf), nb1.reshape(1, _H),
            nW2.astype(bf), nb2.reshape(1, _H),
            nW3.astype(bf), nb3.reshape(1, _D),
            ng.reshape(1, _D), nbt.reshape(1, _D))


def kernel(x, edge_attr, senders, receivers, n_atoms,
           eW1, eb1, eW2, eb2, eW3, eb3, eg, ebt,
           nW1, nb1, nW2, nb2, nW3, nb3, ng, nbt):
    ws = _prep_weights(eW1, eb1, eW2, eb2, eW3, eb3, eg, ebt,
                       nW1, nb1, nW2, nb2, nW3, nb3, ng, nbt)
    gath = _sc_gather()
    m = s = None
    for p in range(_P):
        rec_p, snd_p = gath(x,
                            lax.slice(receivers, (p * _EP,), ((p + 1) * _EP,)),
                            lax.slice(senders, (p * _EP,), ((p + 1) * _EP,)))
        m, s = _tc_edge_part(p, rec_p, snd_p, edge_attr, ws, m, s)
    x_out = _node_call(s, ws)
    return (x_out, m)
